# baseline (speedup 1.0000x reference)
import functools

import jax
import jax.numpy as jnp
from jax import lax
from jax.experimental import pallas as pl
from jax.experimental.pallas import tpu as pltpu
from jax.experimental.pallas import tpu_sc as plsc


def _sc_min(b1):
    mesh = plsc.VectorSubcoreMesh(core_axis_name="c", subcore_axis_name="s",
                                  num_cores=2, num_subcores=16)

    @functools.partial(
        pl.kernel, mesh=mesh,
        compiler_params=pltpu.CompilerParams(needs_layout_passes=False),
        out_type=jax.ShapeDtypeStruct((2,), jnp.float32),
        scratch_types=[pltpu.VMEM((16,), jnp.float32)])
    def k(b1_hbm, out_hbm, buf_v):
        t = lax.axis_index("s")
        c = lax.axis_index("c")

        @pl.when((t == 0) & (c == 0))
        def _():
            pltpu.sync_copy(b1_hbm.at[pl.ds(0, 8)], buf_v.at[pl.ds(0, 8)])
            buf_v[...] = buf_v[...] + 1.0
            pltpu.sync_copy(buf_v.at[pl.ds(0, 2)], out_hbm)

    return k(b1)


def kernel(x, edge_index, W1, b1, fc1_W, fc1_b, fc2_W, fc2_b):
    return _sc_min(b1).reshape(1, 2)
